# baseline (speedup 1.0000x reference)
import functools

import jax
import jax.numpy as jnp
from jax import lax
from jax.experimental import pallas as pl
from jax.experimental.pallas import tpu as pltpu

B = 16384
BLK = 8192
D = 32
V_CENTER, V_SUBJECT, V_GRADE, V_METHOD = 66, 34, 18, 10
OFF_S = V_CENTER
OFF_G = OFF_S + V_SUBJECT
OFF_M = OFF_G + V_GRADE
NCAT = OFF_M + V_METHOD
TEXP_ROWS = 136


def _body(c_ref, s_ref, g_ref, m_ref, cost_ref, time_ref,
          texp_ref, w1_ref, b1_ref, w2_ref, b2_ref, w3_ref, b3_ref,
          out_ref, m_scr):
    @pl.when(pl.program_id(0) == 0)
    def _fold():
        m_scr[...] = jnp.dot(texp_ref[...], w1_ref[...],
                             preferred_element_type=jnp.float32)

    c = c_ref[...]
    s = s_ref[...] + OFF_S
    g = g_ref[...] + OFF_G
    m = m_ref[...] + OFF_M

    col = lax.broadcasted_iota(jnp.int32, (BLK, NCAT), 1)
    onehot = ((col == c[:, None])
              | (col == s[:, None])
              | (col == g[:, None])
              | (col == m[:, None])).astype(jnp.bfloat16)

    oct_ = jnp.concatenate(
        [onehot,
         cost_ref[...].astype(jnp.bfloat16)[:, None],
         time_ref[...].astype(jnp.bfloat16)[:, None]], axis=1)

    mct = m_scr[0:NCAT + 2, :].astype(jnp.bfloat16)
    b1pp = (b1_ref[...][None, :] + m_scr[NCAT + 2:NCAT + 3, :]
            + m_scr[NCAT + 3:NCAT + 4, :])

    h1 = jnp.dot(oct_, mct, preferred_element_type=jnp.float32)
    h1 = jnp.maximum(h1 + b1pp, 0.0)
    h2 = jnp.maximum(
        jnp.dot(h1.astype(jnp.bfloat16), w2_ref[...].astype(jnp.bfloat16),
                preferred_element_type=jnp.float32)
        + b2_ref[...][None, :],
        0.0)
    out_ref[...] = (jnp.dot(h2.astype(jnp.bfloat16),
                            w3_ref[...].astype(jnp.bfloat16),
                            preferred_element_type=jnp.float32)
                    + b3_ref[...][None, :])


def kernel(center_idx, subject_idx, grade_idx, method_idx, cost, time,
           center_table, subject_table, grade_table, method_table,
           cost_W, cost_b, time_W, time_b,
           W1, b1, W2, b2, W3, b3):
    nb = B // BLK
    ci = center_idx.astype(jnp.int32)
    si = subject_idx.astype(jnp.int32)
    gi = grade_idx.astype(jnp.int32)
    mi = method_idx.astype(jnp.int32)

    texp = jnp.zeros((TEXP_ROWS, 192), dtype=jnp.float32)
    texp = texp.at[0:OFF_S, 0:32].set(center_table)
    texp = texp.at[OFF_S:OFF_G, 32:64].set(subject_table)
    texp = texp.at[OFF_G:OFF_M, 64:96].set(grade_table)
    texp = texp.at[OFF_M:NCAT, 96:128].set(method_table)
    texp = texp.at[NCAT, 128:160].set(cost_W[0])
    texp = texp.at[NCAT + 1, 160:192].set(time_W[0])
    texp = texp.at[NCAT + 2, 128:160].set(cost_b)
    texp = texp.at[NCAT + 3, 160:192].set(time_b)

    vec_spec = pl.BlockSpec((BLK,), lambda i: (i,))
    full = lambda a: pl.BlockSpec(a.shape, lambda i: (0,) * a.ndim)

    return pl.pallas_call(
        _body,
        grid=(nb,),
        in_specs=[vec_spec, vec_spec, vec_spec, vec_spec, vec_spec, vec_spec,
                  full(texp), full(W1), full(b1), full(W2), full(b2),
                  full(W3), full(b3)],
        out_specs=pl.BlockSpec((BLK, D), lambda i: (i, 0)),
        out_shape=jax.ShapeDtypeStruct((B, D), jnp.float32),
        scratch_shapes=[pltpu.VMEM((TEXP_ROWS, 256), jnp.float32)],
        compiler_params=pltpu.CompilerParams(
            dimension_semantics=("arbitrary",)),
    )(ci, si, gi, mi, cost, time, texp, W1, b1, W2, b2, W3, b3)
